# baseline (speedup 1.0000x reference)
import functools

import jax
import jax.numpy as jnp
from jax.experimental import pallas as pl

_DIMS = [32, 64, 128, 256]
_K = 16
_RATIO = 0.25


def _input_bn_body(x_ref, w_ref, b_ref, g_ref, be_ref, o_ref):
    t = jnp.dot(x_ref[...], w_ref[...], preferred_element_type=jnp.float32)
    t = t + b_ref[...]
    m = jnp.mean(t, axis=0, keepdims=True)
    v = jnp.mean((t - m) ** 2, axis=0, keepdims=True)
    t = (t - m) * jax.lax.rsqrt(v + 1e-5) * g_ref[...] + be_ref[...]
    o_ref[...] = jnp.maximum(t, 0.0)


def _input_bn(pos, p):
    B, N, _ = pos.shape
    d = p["W"].shape[1]
    x2 = pos.reshape(B * N, 3)
    out = pl.pallas_call(
        _input_bn_body,
        out_shape=jax.ShapeDtypeStruct((B * N, d), jnp.float32),
    )(x2, p["W"], p["b"].reshape(1, d), p["gamma"].reshape(1, d),
      p["beta"].reshape(1, d))
    return out.reshape(B, N, d)


def _linear(p, x):
    return x @ p["W"] + p["b"]


def _bn(p, x, eps=1e-5):
    ax = tuple(range(x.ndim - 1))
    m = jnp.mean(x, axis=ax, keepdims=True)
    v = jnp.var(x, axis=ax, keepdims=True)
    return (x - m) / jnp.sqrt(v + eps) * p["gamma"] + p["beta"]


def _mlp2_apply(p, x):
    return jax.nn.relu(_linear(p["l2"], jax.nn.relu(_linear(p["l1"], x))))


def _pdist2(a, b):
    aa = jnp.sum(a * a, axis=-1)
    bb = jnp.sum(b * b, axis=-1)
    return aa[..., :, None] + bb[..., None, :] - 2.0 * jnp.einsum(
        'bid,bjd->bij', a, b)


def _knn_self(pos, k):
    d = _pdist2(pos, pos)
    n = pos.shape[1]
    d = d + jnp.eye(n, dtype=d.dtype)[None] * 1e10
    _, idx = jax.lax.top_k(-d, k)
    return idx


def _knn_query(qpos, pos, k):
    d = _pdist2(qpos, pos)
    _, idx = jax.lax.top_k(-d, k)
    return idx


def _fps(pos, n_sample):
    def one(p):
        dist = jnp.sum((p - p[0]) ** 2, axis=-1)
        idxs = jnp.zeros((n_sample,), jnp.int32)

        def body(i, c):
            idxs, dist = c
            nxt = jnp.argmax(dist).astype(jnp.int32)
            idxs = idxs.at[i].set(nxt)
            dist = jnp.minimum(dist, jnp.sum((p - p[nxt]) ** 2, axis=-1))
            return (idxs, dist)

        idxs, _ = jax.lax.fori_loop(1, n_sample, body, (idxs, dist))
        return idxs

    return jax.vmap(one)(pos)


def _gather(a, idx):
    return jax.vmap(lambda ab, ib: ab[ib])(a, idx)


def _ptconv(p, x, pos, nbr):
    v = _linear(p["lin"], x)
    a_src = _linear(p["lin_src"], x)
    a_dst = _linear(p["lin_dst"], x)
    x_j = _gather(v, nbr)
    a_j = _gather(a_src, nbr)
    pos_j = _gather(pos, nbr)
    delta = _mlp2_apply(p["pos_nn"], pos[:, :, None, :] - pos_j)
    alpha = a_dst[:, :, None, :] - a_j + delta
    alpha = _mlp2_apply(p["attn_nn"], alpha)
    alpha = jax.nn.softmax(alpha, axis=2)
    return jnp.sum(alpha * (x_j + delta), axis=2)


def _tblock(p, x, pos):
    x = jax.nn.relu(_linear(p["lin_in"], x))
    k = min(_K, pos.shape[1] - 1)
    nbr = _knn_self(pos, k)
    x = _ptconv(p, x, pos, nbr)
    return jax.nn.relu(_linear(p["lin_out"], x))


def _tdown(p, x, pos):
    n = pos.shape[1]
    n_sub = int(n * _RATIO)
    idx = _fps(pos, n_sub)
    sub_pos = _gather(pos, idx)
    nbr = _knn_query(sub_pos, pos, _K)
    xm = jax.nn.relu(_bn(p, _linear(p, x)))
    x_out = jnp.max(_gather(xm, nbr), axis=2)
    return x_out, sub_pos


def kernel(data, params):
    pos = data
    x = _input_bn(pos, params["mlp_input"])
    x = _tblock(params["t_in"], x, pos)
    for i in range(3):
        x, pos = _tdown(params["td"][i], x, pos)
        x = _tblock(params["tb"][i], x, pos)
    return jnp.mean(x, axis=1)


# speedup vs baseline: 8.5087x; 8.5087x over previous
import functools

import jax
import jax.numpy as jnp
from jax import lax
from jax.experimental import pallas as pl
from jax.experimental.pallas import tpu as pltpu
from jax.experimental.pallas import tpu_sc as plsc

_DIMS = [32, 64, 128, 256]
_K = 16
_NC, _NS = 2, 16
_NW = _NC * _NS


def _bn_linear_body(x_ref, w_ref, b_ref, g_ref, be_ref, o_ref):
    t = jnp.dot(x_ref[...], w_ref[...], preferred_element_type=jnp.float32)
    t = t + b_ref[...]
    m = jnp.mean(t, axis=0, keepdims=True)
    v = jnp.mean((t - m) ** 2, axis=0, keepdims=True)
    t = (t - m) * lax.rsqrt(v + 1e-5) * g_ref[...] + be_ref[...]
    o_ref[...] = jnp.maximum(t, 0.0)


def _bn_linear(x2, p):
    m, dout = x2.shape[0], p["W"].shape[1]
    return pl.pallas_call(
        _bn_linear_body,
        out_shape=jax.ShapeDtypeStruct((m, dout), jnp.float32),
    )(x2, p["W"], p["b"].reshape(1, dout), p["gamma"].reshape(1, dout),
      p["beta"].reshape(1, dout))


def _fps_body(px_ref, py_ref, pz_ref, sx_ref, sy_ref, sz_ref, *, n_sub):
    pxt = px_ref[...].T
    pyt = py_ref[...].T
    pzt = pz_ref[...].T
    sx_ref[0:1, :] = pxt[0:1, :]
    sy_ref[0:1, :] = pyt[0:1, :]
    sz_ref[0:1, :] = pzt[0:1, :]
    n = pxt.shape[0]
    riota = lax.broadcasted_iota(jnp.int32, pxt.shape, 0)
    dist = ((pxt - pxt[0:1, :]) ** 2 + (pyt - pyt[0:1, :]) ** 2
            + (pzt - pzt[0:1, :]) ** 2)

    def body(i, dist):
        nxt = jnp.argmax(dist, axis=0).astype(jnp.int32)
        msel = riota == nxt[None, :]
        cx = jnp.sum(jnp.where(msel, pxt, 0.0), axis=0)
        cy = jnp.sum(jnp.where(msel, pyt, 0.0), axis=0)
        cz = jnp.sum(jnp.where(msel, pzt, 0.0), axis=0)
        sx_ref[pl.ds(i, 1), :] = cx[None, :]
        sy_ref[pl.ds(i, 1), :] = cy[None, :]
        sz_ref[pl.ds(i, 1), :] = cz[None, :]
        nd = ((pxt - cx[None, :]) ** 2 + (pyt - cy[None, :]) ** 2
              + (pzt - cz[None, :]) ** 2)
        return jnp.minimum(dist, nd)

    lax.fori_loop(1, n_sub, body, dist)


def _fps(px, py, pz, n_sub):
    b, _, n = px.shape
    shp = jax.ShapeDtypeStruct((n_sub, b), jnp.float32)
    return pl.pallas_call(
        functools.partial(_fps_body, n_sub=n_sub),
        out_shape=[shp, shp, shp],
    )(px.reshape(b, n), py.reshape(b, n), pz.reshape(b, n))


def _knn_body(qx_ref, qy_ref, qz_ref, px_ref, py_ref, pz_ref, o_ref, *,
              rb, n, self_ex):
    b = pl.program_id(0)
    r = pl.program_id(1)
    qx = qx_ref[0, 0, :].reshape(rb, 1)
    qy = qy_ref[0, 0, :].reshape(rb, 1)
    qz = qz_ref[0, 0, :].reshape(rb, 1)
    pxv = px_ref[0, 0, :].reshape(1, n)
    pyv = py_ref[0, 0, :].reshape(1, n)
    pzv = pz_ref[0, 0, :].reshape(1, n)
    qq = qx * qx + qy * qy + qz * qz
    pp = pxv * pxv + pyv * pyv + pzv * pzv
    q = jnp.concatenate([qx, qy, qz], axis=1)
    pt = jnp.concatenate([pxv, pyv, pzv], axis=0)
    ab = jnp.dot(q, pt, preferred_element_type=jnp.float32)
    d = qq + pp - 2.0 * ab
    ciota = lax.broadcasted_iota(jnp.int32, (rb, n), 1)
    if self_ex:
        riota = lax.broadcasted_iota(jnp.int32, (rb, n), 0) + r * rb
        d = jnp.where(ciota == riota, d + 1e10, d)
    base = b * n
    for kk in range(_K):
        am = jnp.argmin(d, axis=1).astype(jnp.int32)
        o_ref[:, kk:kk + 1] = am[:, None] + base
        d = jnp.where(ciota == am[:, None], 1e30, d)


def _knn(q3, p3, self_ex):
    b, _, nq = q3[0].shape
    n = p3[0].shape[2]
    rb = min(512, nq)
    nqb = nq // rb
    qspec = pl.BlockSpec((1, 1, rb), lambda bb, rr: (bb, 0, rr))
    pspec = pl.BlockSpec((1, 1, n), lambda bb, rr: (bb, 0, 0))
    return pl.pallas_call(
        functools.partial(_knn_body, rb=rb, n=n, self_ex=self_ex),
        grid=(b, nqb),
        in_specs=[qspec, qspec, qspec, pspec, pspec, pspec],
        out_specs=pl.BlockSpec((rb, _K), lambda bb, rr: (bb * nqb + rr, 0)),
        out_shape=jax.ShapeDtypeStruct((b * nq, _K), jnp.int32),
    )(*q3, *p3)


@functools.lru_cache(maxsize=None)
def _sc_gather_build(d_cols, n_gather):
    b_per_w = n_gather // _NW
    c = min(128, b_per_w)
    nch = b_per_w // c
    mesh = plsc.VectorSubcoreMesh(core_axis_name="c", subcore_axis_name="s",
                                  num_cores=_NC, num_subcores=_NS)

    @functools.partial(
        pl.kernel,
        out_type=jax.ShapeDtypeStruct((n_gather, d_cols), jnp.float32),
        mesh=mesh,
        scratch_types=[pltpu.VMEM((nch, c), jnp.int32),
                       pltpu.VMEM((c, d_cols), jnp.float32),
                       pltpu.SemaphoreType.DMA],
        compiler_params=pltpu.CompilerParams(use_tc_tiling_on_sc=False),
    )
    def gather_k(table_hbm, idx_hbm, out_hbm, idx_v, rows_v, sem):
        wid = lax.axis_index("s") * _NC + lax.axis_index("c")
        pltpu.sync_copy(idx_hbm.at[pl.ds(wid * nch, nch)], idx_v)

        def chunk(j, carry):
            pltpu.async_copy(table_hbm.at[idx_v.at[j]], rows_v, sem).wait()
            pltpu.sync_copy(rows_v, out_hbm.at[pl.ds((wid * nch + j) * c, c)])
            return carry

        lax.fori_loop(0, nch, chunk, 0)

    return gather_k, c


def _sc_gather(table, idx_flat):
    n_gather = idx_flat.shape[0]
    k, c = _sc_gather_build(table.shape[1], n_gather)
    return k(table, idx_flat.reshape(n_gather // c, c))


def _pre_body(xin_ref, pos_ref, wi_ref, bi_ref, wl_ref, bl_ref, ws_ref,
              bs_ref, wd_ref, bd_ref, ot_ref, oa_ref, *, m, d, pool):
    x = xin_ref[...]
    if pool:
        x = jnp.max(x.reshape(m, _K, d), axis=1)
    h = jnp.maximum(
        jnp.dot(x, wi_ref[...], preferred_element_type=jnp.float32)
        + bi_ref[...], 0.0)
    v = jnp.dot(h, wl_ref[...], preferred_element_type=jnp.float32) + bl_ref[...]
    a_src = jnp.dot(h, ws_ref[...], preferred_element_type=jnp.float32) + bs_ref[...]
    a_dst = jnp.dot(h, wd_ref[...], preferred_element_type=jnp.float32) + bd_ref[...]
    ot_ref[...] = jnp.concatenate(
        [v, a_src, pos_ref[...], jnp.zeros((m, 13), jnp.float32)], axis=1)
    oa_ref[...] = a_dst


def _pre(p, xin, pos3, m, d, pool):
    dt = 2 * d + 16
    r1 = lambda a: a.reshape(1, -1)
    return pl.pallas_call(
        functools.partial(_pre_body, m=m, d=d, pool=pool),
        out_shape=[jax.ShapeDtypeStruct((m, dt), jnp.float32),
                   jax.ShapeDtypeStruct((m, d), jnp.float32)],
    )(xin, pos3, p["lin_in"]["W"], r1(p["lin_in"]["b"]),
      p["lin"]["W"], r1(p["lin"]["b"]),
      p["lin_src"]["W"], r1(p["lin_src"]["b"]),
      p["lin_dst"]["W"], r1(p["lin_dst"]["b"]))


def _post_body(g_ref, ad_ref, pos_ref, w1p_ref, b1p_ref, w2p_ref, b2p_ref,
               w1a_ref, b1a_ref, w2a_ref, b2a_ref, wo_ref, bo_ref, o_ref, *,
               r, d, nb, final_mean):
    g = g_ref[...]
    vj = g[:, :d]
    aj = g[:, d:2 * d]
    pj = g[:, 2 * d:2 * d + 3]
    pd = (pos_ref[...][:, None, :] - pj.reshape(r, _K, 3)).reshape(r * _K, 3)
    t = jnp.maximum(
        jnp.dot(pd, w1p_ref[...], preferred_element_type=jnp.float32)
        + b1p_ref[...], 0.0)
    delta = jnp.maximum(
        jnp.dot(t, w2p_ref[...], preferred_element_type=jnp.float32)
        + b2p_ref[...], 0.0)
    d3 = delta.reshape(r, _K, d)
    al = (ad_ref[...][:, None, :] - aj.reshape(r, _K, d) + d3).reshape(r * _K, d)
    t2 = jnp.maximum(
        jnp.dot(al, w1a_ref[...], preferred_element_type=jnp.float32)
        + b1a_ref[...], 0.0)
    alpha = jnp.maximum(
        jnp.dot(t2, w2a_ref[...], preferred_element_type=jnp.float32)
        + b2a_ref[...], 0.0).reshape(r, _K, d)
    mx = jnp.max(alpha, axis=1, keepdims=True)
    e = jnp.exp(alpha - mx)
    w = e / jnp.sum(e, axis=1, keepdims=True)
    outv = jnp.sum(w * (vj.reshape(r, _K, d) + d3), axis=1)
    y = jnp.maximum(
        jnp.dot(outv, wo_ref[...], preferred_element_type=jnp.float32)
        + bo_ref[...], 0.0)
    if final_mean:
        o_ref[...] = jnp.mean(y.reshape(nb, r // nb, d), axis=1)
    else:
        o_ref[...] = y


def _post(p, g, a_dst, pos3, m, d, nb, final_mean):
    dt = 2 * d + 16
    r = min(512, m)
    ng = m // r
    if final_mean:
        r, ng = m, 1
        out_shape = jax.ShapeDtypeStruct((nb, d), jnp.float32)
        out_spec = pl.BlockSpec((nb, d), lambda i: (0, 0))
    else:
        out_shape = jax.ShapeDtypeStruct((m, d), jnp.float32)
        out_spec = pl.BlockSpec((r, d), lambda i: (i, 0))
    full = lambda s: pl.BlockSpec(s, lambda i: tuple(0 for _ in s))
    r1 = lambda a: a.reshape(1, -1)
    return pl.pallas_call(
        functools.partial(_post_body, r=r, d=d, nb=nb, final_mean=final_mean),
        grid=(ng,),
        in_specs=[pl.BlockSpec((r * _K, dt), lambda i: (i, 0)),
                  pl.BlockSpec((r, d), lambda i: (i, 0)),
                  pl.BlockSpec((r, 3), lambda i: (i, 0)),
                  full((3, 64)), full((1, 64)), full((64, d)), full((1, d)),
                  full((d, 64)), full((1, 64)), full((64, d)), full((1, d)),
                  full((d, d)), full((1, d))],
        out_specs=out_spec,
        out_shape=out_shape,
    )(g, a_dst, pos3,
      p["pos_nn"]["l1"]["W"], r1(p["pos_nn"]["l1"]["b"]),
      p["pos_nn"]["l2"]["W"], r1(p["pos_nn"]["l2"]["b"]),
      p["attn_nn"]["l1"]["W"], r1(p["attn_nn"]["l1"]["b"]),
      p["attn_nn"]["l2"]["W"], r1(p["attn_nn"]["l2"]["b"]),
      p["lin_out"]["W"], r1(p["lin_out"]["b"]))


def _tblock(p, xin, pos3, nbr, m, d, nb, pool, final_mean):
    table, a_dst = _pre(p, xin, pos3, m, d, pool)
    g = _sc_gather(table, nbr.reshape(-1))
    return _post(p, g, a_dst, pos3, m, d, nb, final_mean)


def kernel(data, params):
    b, n, _ = data.shape
    pos3 = data.reshape(b * n, 3)
    p3 = (data[:, :, 0].reshape(b, 1, n), data[:, :, 1].reshape(b, 1, n),
          data[:, :, 2].reshape(b, 1, n))

    x = _bn_linear(pos3, params["mlp_input"])
    nbr = _knn(p3, p3, self_ex=True)
    x = _tblock(params["t_in"], x, pos3, nbr, b * n, _DIMS[0], b,
                pool=False, final_mean=False)

    for i in range(3):
        n_sub = n // 4
        d_out = _DIMS[i + 1]
        sx, sy, sz = _fps(*p3, n_sub)
        q3 = (sx.T.reshape(b, 1, n_sub), sy.T.reshape(b, 1, n_sub),
              sz.T.reshape(b, 1, n_sub))
        sub_pos3 = jnp.stack([sx.T, sy.T, sz.T], axis=-1).reshape(b * n_sub, 3)
        nbr_q = _knn(q3, p3, self_ex=False)
        xm = _bn_linear(x, params["td"][i])
        gm = _sc_gather(xm, nbr_q.reshape(-1))
        nbr_s = _knn(q3, q3, self_ex=True)
        x = _tblock(params["tb"][i], gm, sub_pos3, nbr_s, b * n_sub, d_out,
                    b, pool=True, final_mean=(i == 2))
        p3, n, pos3 = q3, n_sub, sub_pos3

    return x


# speedup vs baseline: 13.4079x; 1.5758x over previous
import functools

import jax
import jax.numpy as jnp
from jax import lax
from jax.experimental import pallas as pl
from jax.experimental.pallas import tpu as pltpu
from jax.experimental.pallas import tpu_sc as plsc

_DIMS = [32, 64, 128, 256]
_K = 16
_NC, _NS = 2, 16
_NW = _NC * _NS


def _bn_linear_body(x_ref, w_ref, b_ref, g_ref, be_ref, o_ref):
    t = jnp.dot(x_ref[...], w_ref[...], preferred_element_type=jnp.float32)
    t = t + b_ref[...]
    m = jnp.mean(t, axis=0, keepdims=True)
    v = jnp.mean((t - m) ** 2, axis=0, keepdims=True)
    t = (t - m) * lax.rsqrt(v + 1e-5) * g_ref[...] + be_ref[...]
    o_ref[...] = jnp.maximum(t, 0.0)


def _bn_linear(x2, p):
    m, dout = x2.shape[0], p["W"].shape[1]
    return pl.pallas_call(
        _bn_linear_body,
        out_shape=jax.ShapeDtypeStruct((m, dout), jnp.float32),
    )(x2, p["W"], p["b"].reshape(1, dout), p["gamma"].reshape(1, dout),
      p["beta"].reshape(1, dout))


def _fps_body(px_ref, py_ref, pz_ref, sx_ref, sy_ref, sz_ref, *, b, rr,
              n_sub):
    px3 = px_ref[...].reshape(b, rr, 128)
    py3 = py_ref[...].reshape(b, rr, 128)
    pz3 = pz_ref[...].reshape(b, rr, 128)
    sx_ref[0:1, :] = px3[:, 0:1, 0:1].reshape(1, b)
    sy_ref[0:1, :] = py3[:, 0:1, 0:1].reshape(1, b)
    sz_ref[0:1, :] = pz3[:, 0:1, 0:1].reshape(1, b)
    pid = (lax.broadcasted_iota(jnp.int32, (b, rr, 128), 1) * 128
           + lax.broadcasted_iota(jnp.int32, (b, rr, 128), 2))
    dist = ((px3 - px3[:, 0:1, 0:1]) ** 2 + (py3 - py3[:, 0:1, 0:1]) ** 2
            + (pz3 - pz3[:, 0:1, 0:1]) ** 2)

    def body(i, dist):
        m = jnp.max(jnp.max(dist, axis=2, keepdims=True), axis=1,
                    keepdims=True)
        cand = jnp.where(dist == m, pid, jnp.int32(2 ** 30))
        nxt = jnp.min(jnp.min(cand, axis=2, keepdims=True), axis=1,
                      keepdims=True)
        msel = pid == nxt
        cx = jnp.sum(jnp.sum(jnp.where(msel, px3, 0.0), axis=2,
                             keepdims=True), axis=1, keepdims=True)
        cy = jnp.sum(jnp.sum(jnp.where(msel, py3, 0.0), axis=2,
                             keepdims=True), axis=1, keepdims=True)
        cz = jnp.sum(jnp.sum(jnp.where(msel, pz3, 0.0), axis=2,
                             keepdims=True), axis=1, keepdims=True)
        sx_ref[pl.ds(i, 1), :] = cx.reshape(1, b)
        sy_ref[pl.ds(i, 1), :] = cy.reshape(1, b)
        sz_ref[pl.ds(i, 1), :] = cz.reshape(1, b)
        nd = (px3 - cx) ** 2 + (py3 - cy) ** 2 + (pz3 - cz) ** 2
        return jnp.minimum(dist, nd)

    lax.fori_loop(1, n_sub, body, dist)


def _fps(px, py, pz, n_sub):
    b, _, n = px.shape
    rr = n // 128
    shp = jax.ShapeDtypeStruct((n_sub, b), jnp.float32)
    return pl.pallas_call(
        functools.partial(_fps_body, b=b, rr=rr, n_sub=n_sub),
        out_shape=[shp, shp, shp],
    )(px.reshape(b * rr, 128), py.reshape(b * rr, 128),
      pz.reshape(b * rr, 128))


def _knn_body(qx_ref, qy_ref, qz_ref, px_ref, py_ref, pz_ref, o_ref, *,
              rb, n, self_ex):
    b = pl.program_id(0)
    r = pl.program_id(1)
    qx = qx_ref[0, 0, :].reshape(rb, 1)
    qy = qy_ref[0, 0, :].reshape(rb, 1)
    qz = qz_ref[0, 0, :].reshape(rb, 1)
    pxv = px_ref[0, 0, :].reshape(1, n)
    pyv = py_ref[0, 0, :].reshape(1, n)
    pzv = pz_ref[0, 0, :].reshape(1, n)
    qq = qx * qx + qy * qy + qz * qz
    pp = pxv * pxv + pyv * pyv + pzv * pzv
    q = jnp.concatenate([qx, qy, qz], axis=1)
    pt = jnp.concatenate([pxv, pyv, pzv], axis=0)
    ab = jnp.dot(q, pt, preferred_element_type=jnp.float32)
    d = qq + pp - 2.0 * ab
    ciota = lax.broadcasted_iota(jnp.int32, (rb, n), 1)
    if self_ex:
        riota = lax.broadcasted_iota(jnp.int32, (rb, n), 0) + r * rb
        d = jnp.where(ciota == riota, d + 1e10, d)
    base = b * n
    for kk in range(_K):
        am = jnp.argmin(d, axis=1).astype(jnp.int32)
        o_ref[:, kk:kk + 1] = am[:, None] + base
        d = jnp.where(ciota == am[:, None], 1e30, d)


def _knn(q3, p3, self_ex):
    b, _, nq = q3[0].shape
    n = p3[0].shape[2]
    rb = min(512, nq)
    nqb = nq // rb
    qspec = pl.BlockSpec((1, 1, rb), lambda bb, rr: (bb, 0, rr))
    pspec = pl.BlockSpec((1, 1, n), lambda bb, rr: (bb, 0, 0))
    return pl.pallas_call(
        functools.partial(_knn_body, rb=rb, n=n, self_ex=self_ex),
        grid=(b, nqb),
        in_specs=[qspec, qspec, qspec, pspec, pspec, pspec],
        out_specs=pl.BlockSpec((rb, _K), lambda bb, rr: (bb * nqb + rr, 0)),
        out_shape=jax.ShapeDtypeStruct((b * nq, _K), jnp.int32),
    )(*q3, *p3)


@functools.lru_cache(maxsize=None)
def _sc_gather_build(d_cols, n_gather):
    b_per_w = n_gather // _NW
    c = min(128, b_per_w)
    nch = b_per_w // c
    mesh = plsc.VectorSubcoreMesh(core_axis_name="c", subcore_axis_name="s",
                                  num_cores=_NC, num_subcores=_NS)

    @functools.partial(
        pl.kernel,
        out_type=jax.ShapeDtypeStruct((n_gather, d_cols), jnp.float32),
        mesh=mesh,
        scratch_types=[pltpu.VMEM((nch, c), jnp.int32),
                       pltpu.VMEM((c, d_cols), jnp.float32),
                       pltpu.SemaphoreType.DMA],
        compiler_params=pltpu.CompilerParams(use_tc_tiling_on_sc=False),
    )
    def gather_k(table_hbm, idx_hbm, out_hbm, idx_v, rows_v, sem):
        wid = lax.axis_index("s") * _NC + lax.axis_index("c")
        pltpu.sync_copy(idx_hbm.at[pl.ds(wid * nch, nch)], idx_v)

        def chunk(j, carry):
            pltpu.async_copy(table_hbm.at[idx_v.at[j]], rows_v, sem).wait()
            pltpu.sync_copy(rows_v, out_hbm.at[pl.ds((wid * nch + j) * c, c)])
            return carry

        lax.fori_loop(0, nch, chunk, 0)

    return gather_k, c


def _sc_gather(table, idx_flat):
    n_gather = idx_flat.shape[0]
    k, c = _sc_gather_build(table.shape[1], n_gather)
    return k(table, idx_flat.reshape(n_gather // c, c))


def _pre_body(xin_ref, pos_ref, wi_ref, bi_ref, wl_ref, bl_ref, ws_ref,
              bs_ref, wd_ref, bd_ref, ot_ref, oa_ref, *, m, d, pool):
    x = xin_ref[...]
    if pool:
        x = jnp.max(x.reshape(m, _K, d), axis=1)
    h = jnp.maximum(
        jnp.dot(x, wi_ref[...], preferred_element_type=jnp.float32)
        + bi_ref[...], 0.0)
    v = jnp.dot(h, wl_ref[...], preferred_element_type=jnp.float32) + bl_ref[...]
    a_src = jnp.dot(h, ws_ref[...], preferred_element_type=jnp.float32) + bs_ref[...]
    a_dst = jnp.dot(h, wd_ref[...], preferred_element_type=jnp.float32) + bd_ref[...]
    ot_ref[...] = jnp.concatenate(
        [v, a_src, pos_ref[...], jnp.zeros((m, 13), jnp.float32)], axis=1)
    oa_ref[...] = a_dst


def _pre(p, xin, pos3, m, d, pool):
    dt = 2 * d + 16
    r1 = lambda a: a.reshape(1, -1)
    return pl.pallas_call(
        functools.partial(_pre_body, m=m, d=d, pool=pool),
        out_shape=[jax.ShapeDtypeStruct((m, dt), jnp.float32),
                   jax.ShapeDtypeStruct((m, d), jnp.float32)],
    )(xin, pos3, p["lin_in"]["W"], r1(p["lin_in"]["b"]),
      p["lin"]["W"], r1(p["lin"]["b"]),
      p["lin_src"]["W"], r1(p["lin_src"]["b"]),
      p["lin_dst"]["W"], r1(p["lin_dst"]["b"]))


def _post_body(g_ref, ad_ref, pos_ref, w1p_ref, b1p_ref, w2p_ref, b2p_ref,
               w1a_ref, b1a_ref, w2a_ref, b2a_ref, wo_ref, bo_ref, o_ref, *,
               r, d, nb, final_mean):
    g = g_ref[...]
    vj = g[:, :d]
    aj = g[:, d:2 * d]
    pj = g[:, 2 * d:2 * d + 3]
    pd = (pos_ref[...][:, None, :] - pj.reshape(r, _K, 3)).reshape(r * _K, 3)
    t = jnp.maximum(
        jnp.dot(pd, w1p_ref[...], preferred_element_type=jnp.float32)
        + b1p_ref[...], 0.0)
    delta = jnp.maximum(
        jnp.dot(t, w2p_ref[...], preferred_element_type=jnp.float32)
        + b2p_ref[...], 0.0)
    d3 = delta.reshape(r, _K, d)
    al = (ad_ref[...][:, None, :] - aj.reshape(r, _K, d) + d3).reshape(r * _K, d)
    t2 = jnp.maximum(
        jnp.dot(al, w1a_ref[...], preferred_element_type=jnp.float32)
        + b1a_ref[...], 0.0)
    alpha = jnp.maximum(
        jnp.dot(t2, w2a_ref[...], preferred_element_type=jnp.float32)
        + b2a_ref[...], 0.0).reshape(r, _K, d)
    mx = jnp.max(alpha, axis=1, keepdims=True)
    e = jnp.exp(alpha - mx)
    w = e / jnp.sum(e, axis=1, keepdims=True)
    outv = jnp.sum(w * (vj.reshape(r, _K, d) + d3), axis=1)
    y = jnp.maximum(
        jnp.dot(outv, wo_ref[...], preferred_element_type=jnp.float32)
        + bo_ref[...], 0.0)
    if final_mean:
        o_ref[...] = jnp.mean(y.reshape(nb, r // nb, d), axis=1)
    else:
        o_ref[...] = y


def _post(p, g, a_dst, pos3, m, d, nb, final_mean):
    dt = 2 * d + 16
    r = min(512, m)
    ng = m // r
    if final_mean:
        r, ng = m, 1
        out_shape = jax.ShapeDtypeStruct((nb, d), jnp.float32)
        out_spec = pl.BlockSpec((nb, d), lambda i: (0, 0))
    else:
        out_shape = jax.ShapeDtypeStruct((m, d), jnp.float32)
        out_spec = pl.BlockSpec((r, d), lambda i: (i, 0))
    full = lambda s: pl.BlockSpec(s, lambda i: tuple(0 for _ in s))
    r1 = lambda a: a.reshape(1, -1)
    return pl.pallas_call(
        functools.partial(_post_body, r=r, d=d, nb=nb, final_mean=final_mean),
        grid=(ng,),
        in_specs=[pl.BlockSpec((r * _K, dt), lambda i: (i, 0)),
                  pl.BlockSpec((r, d), lambda i: (i, 0)),
                  pl.BlockSpec((r, 3), lambda i: (i, 0)),
                  full((3, 64)), full((1, 64)), full((64, d)), full((1, d)),
                  full((d, 64)), full((1, 64)), full((64, d)), full((1, d)),
                  full((d, d)), full((1, d))],
        out_specs=out_spec,
        out_shape=out_shape,
    )(g, a_dst, pos3,
      p["pos_nn"]["l1"]["W"], r1(p["pos_nn"]["l1"]["b"]),
      p["pos_nn"]["l2"]["W"], r1(p["pos_nn"]["l2"]["b"]),
      p["attn_nn"]["l1"]["W"], r1(p["attn_nn"]["l1"]["b"]),
      p["attn_nn"]["l2"]["W"], r1(p["attn_nn"]["l2"]["b"]),
      p["lin_out"]["W"], r1(p["lin_out"]["b"]))


def _tblock(p, xin, pos3, nbr, m, d, nb, pool, final_mean):
    table, a_dst = _pre(p, xin, pos3, m, d, pool)
    g = _sc_gather(table, nbr.reshape(-1))
    return _post(p, g, a_dst, pos3, m, d, nb, final_mean)


def kernel(data, params):
    b, n, _ = data.shape
    pos3 = data.reshape(b * n, 3)
    p3 = (data[:, :, 0].reshape(b, 1, n), data[:, :, 1].reshape(b, 1, n),
          data[:, :, 2].reshape(b, 1, n))

    x = _bn_linear(pos3, params["mlp_input"])
    nbr = _knn(p3, p3, self_ex=True)
    x = _tblock(params["t_in"], x, pos3, nbr, b * n, _DIMS[0], b,
                pool=False, final_mean=False)

    for i in range(3):
        n_sub = n // 4
        d_out = _DIMS[i + 1]
        sx, sy, sz = _fps(*p3, n_sub)
        q3 = (sx.T.reshape(b, 1, n_sub), sy.T.reshape(b, 1, n_sub),
              sz.T.reshape(b, 1, n_sub))
        sub_pos3 = jnp.stack([sx.T, sy.T, sz.T], axis=-1).reshape(b * n_sub, 3)
        nbr_q = _knn(q3, p3, self_ex=False)
        xm = _bn_linear(x, params["td"][i])
        gm = _sc_gather(xm, nbr_q.reshape(-1))
        nbr_s = _knn(q3, q3, self_ex=True)
        x = _tblock(params["tb"][i], gm, sub_pos3, nbr_s, b * n_sub, d_out,
                    b, pool=True, final_mean=(i == 2))
        p3, n, pos3 = q3, n_sub, sub_pos3

    return x


# speedup vs baseline: 14.7418x; 1.0995x over previous
import functools

import jax
import jax.numpy as jnp
from jax import lax
from jax.experimental import pallas as pl
from jax.experimental.pallas import tpu as pltpu
from jax.experimental.pallas import tpu_sc as plsc

_DIMS = [32, 64, 128, 256]
_K = 16
_NC, _NS = 2, 16
_NW = _NC * _NS


def _bn_linear_body(x_ref, w_ref, b_ref, g_ref, be_ref, o_ref, *, pad):
    t = jnp.dot(x_ref[...], w_ref[...], preferred_element_type=jnp.float32)
    t = t + b_ref[...]
    m = jnp.mean(t, axis=0, keepdims=True)
    v = jnp.mean((t - m) ** 2, axis=0, keepdims=True)
    t = (t - m) * lax.rsqrt(v + 1e-5) * g_ref[...] + be_ref[...]
    t = jnp.maximum(t, 0.0)
    if pad:
        t = jnp.concatenate(
            [t, jnp.zeros((t.shape[0], pad), jnp.float32)], axis=1)
    o_ref[...] = t


def _bn_linear(x2, p, pad_to=None):
    m, dout = x2.shape[0], p["W"].shape[1]
    dp = dout if pad_to is None else pad_to
    return pl.pallas_call(
        functools.partial(_bn_linear_body, pad=dp - dout),
        out_shape=jax.ShapeDtypeStruct((m, dp), jnp.float32),
    )(x2, p["W"], p["b"].reshape(1, dout), p["gamma"].reshape(1, dout),
      p["beta"].reshape(1, dout))


def _fps_body(px_ref, py_ref, pz_ref, sx_ref, sy_ref, sz_ref, *, b, rr,
              n_sub):
    px3 = px_ref[...].reshape(b, rr, 128)
    py3 = py_ref[...].reshape(b, rr, 128)
    pz3 = pz_ref[...].reshape(b, rr, 128)
    sx_ref[0:1, :] = px3[:, 0:1, 0:1].reshape(1, b)
    sy_ref[0:1, :] = py3[:, 0:1, 0:1].reshape(1, b)
    sz_ref[0:1, :] = pz3[:, 0:1, 0:1].reshape(1, b)
    pid = (lax.broadcasted_iota(jnp.int32, (b, rr, 128), 1) * 128
           + lax.broadcasted_iota(jnp.int32, (b, rr, 128), 2))
    dist = ((px3 - px3[:, 0:1, 0:1]) ** 2 + (py3 - py3[:, 0:1, 0:1]) ** 2
            + (pz3 - pz3[:, 0:1, 0:1]) ** 2)

    def body(i, dist):
        m = jnp.max(jnp.max(dist, axis=2, keepdims=True), axis=1,
                    keepdims=True)
        cand = jnp.where(dist == m, pid, jnp.int32(2 ** 30))
        nxt = jnp.min(jnp.min(cand, axis=2, keepdims=True), axis=1,
                      keepdims=True)
        msel = pid == nxt
        cx = jnp.sum(jnp.sum(jnp.where(msel, px3, 0.0), axis=2,
                             keepdims=True), axis=1, keepdims=True)
        cy = jnp.sum(jnp.sum(jnp.where(msel, py3, 0.0), axis=2,
                             keepdims=True), axis=1, keepdims=True)
        cz = jnp.sum(jnp.sum(jnp.where(msel, pz3, 0.0), axis=2,
                             keepdims=True), axis=1, keepdims=True)
        sx_ref[pl.ds(i, 1), :] = cx.reshape(1, b)
        sy_ref[pl.ds(i, 1), :] = cy.reshape(1, b)
        sz_ref[pl.ds(i, 1), :] = cz.reshape(1, b)
        nd = (px3 - cx) ** 2 + (py3 - cy) ** 2 + (pz3 - cz) ** 2
        return jnp.minimum(dist, nd)

    lax.fori_loop(1, n_sub, body, dist)


def _fps(px, py, pz, n_sub):
    b, _, n = px.shape
    rr = n // 128
    shp = jax.ShapeDtypeStruct((n_sub, b), jnp.float32)
    return pl.pallas_call(
        functools.partial(_fps_body, b=b, rr=rr, n_sub=n_sub),
        out_shape=[shp, shp, shp],
    )(px.reshape(b * rr, 128), py.reshape(b * rr, 128),
      pz.reshape(b * rr, 128))


def _knn_body(qx_ref, qy_ref, qz_ref, px_ref, py_ref, pz_ref, o_ref, *,
              rb, n, self_ex):
    b = pl.program_id(0)
    r = pl.program_id(1)
    qx = qx_ref[0, 0, :].reshape(rb, 1)
    qy = qy_ref[0, 0, :].reshape(rb, 1)
    qz = qz_ref[0, 0, :].reshape(rb, 1)
    pxv = px_ref[0, 0, :].reshape(1, n)
    pyv = py_ref[0, 0, :].reshape(1, n)
    pzv = pz_ref[0, 0, :].reshape(1, n)
    qq = qx * qx + qy * qy + qz * qz
    pp = pxv * pxv + pyv * pyv + pzv * pzv
    q = jnp.concatenate([qx, qy, qz], axis=1)
    pt = jnp.concatenate([pxv, pyv, pzv], axis=0)
    ab = jnp.dot(q, pt, preferred_element_type=jnp.float32)
    d = qq + pp - 2.0 * ab
    ciota = lax.broadcasted_iota(jnp.int32, (rb, n), 1)
    if self_ex:
        riota = lax.broadcasted_iota(jnp.int32, (rb, n), 0) + r * rb
        d = jnp.where(ciota == riota, d + 1e10, d)
    base = b * n
    cols = []
    for kk in range(_K):
        am = jnp.argmin(d, axis=1).astype(jnp.int32)
        cols.append(am[:, None] + base)
        if kk + 1 < _K:
            d = jnp.where(ciota == am[:, None], 1e30, d)
    o_ref[...] = jnp.concatenate(cols, axis=1)


def _knn(q3, p3, self_ex):
    b, _, nq = q3[0].shape
    n = p3[0].shape[2]
    rb = min(512, nq)
    nqb = nq // rb
    qspec = pl.BlockSpec((1, 1, rb), lambda bb, rr: (bb, 0, rr))
    pspec = pl.BlockSpec((1, 1, n), lambda bb, rr: (bb, 0, 0))
    out = pl.pallas_call(
        functools.partial(_knn_body, rb=rb, n=n, self_ex=self_ex),
        grid=(b, nqb),
        in_specs=[qspec, qspec, qspec, pspec, pspec, pspec],
        out_specs=pl.BlockSpec((rb, _K), lambda bb, rr: (bb * nqb + rr, 0)),
        out_shape=jax.ShapeDtypeStruct((b * nq, _K), jnp.int32),
    )(*q3, *p3)
    return out.reshape(b * nq // 8, 128)


_C = 128


@functools.lru_cache(maxsize=None)
def _sc_gather_build(d_cols, n_gather):
    tc = n_gather // _C
    nch = max(1, tc // _NW)
    mesh = plsc.VectorSubcoreMesh(core_axis_name="c", subcore_axis_name="s",
                                  num_cores=_NC, num_subcores=_NS)

    @functools.partial(
        pl.kernel,
        out_type=jax.ShapeDtypeStruct((n_gather, d_cols), jnp.float32),
        mesh=mesh,
        scratch_types=[pltpu.VMEM((nch, _C), jnp.int32),
                       pltpu.VMEM((_C, d_cols), jnp.float32),
                       pltpu.VMEM((_C, d_cols), jnp.float32),
                       pltpu.SemaphoreType.DMA,
                       pltpu.SemaphoreType.DMA],
        compiler_params=pltpu.CompilerParams(use_tc_tiling_on_sc=False),
    )
    def gather_k(table_hbm, idx_hbm, out_hbm, idx_v, rows_a, rows_b,
                 sem_g, sem_o):
        wid = lax.axis_index("s") * _NC + lax.axis_index("c")
        base = wid * nch

        @pl.when(base < tc)
        def _():
            pltpu.sync_copy(idx_hbm.at[pl.ds(base, nch)], idx_v)
            if nch == 1:
                pltpu.async_copy(table_hbm.at[idx_v.at[0]], rows_a,
                                 sem_g).wait()
                pltpu.sync_copy(rows_a, out_hbm.at[pl.ds(base * _C, _C)])
            else:
                def pair(j2, carry):
                    l0 = 2 * j2
                    g0 = pltpu.async_copy(table_hbm.at[idx_v.at[l0]],
                                          rows_a, sem_g)
                    g1 = pltpu.async_copy(table_hbm.at[idx_v.at[l0 + 1]],
                                          rows_b, sem_g)
                    g0.wait()
                    o0 = pltpu.async_copy(
                        rows_a, out_hbm.at[pl.ds((base + l0) * _C, _C)],
                        sem_o)
                    g1.wait()
                    o1 = pltpu.async_copy(
                        rows_b, out_hbm.at[pl.ds((base + l0 + 1) * _C, _C)],
                        sem_o)
                    o0.wait()
                    o1.wait()
                    return carry

                lax.fori_loop(0, nch // 2, pair, 0)

    return gather_k


def _sc_gather(table, idx2d):
    n_gather = idx2d.shape[0] * idx2d.shape[1]
    k = _sc_gather_build(table.shape[1], n_gather)
    return k(table, idx2d)


def _pad_up(c):
    return (c + 127) // 128 * 128


def _pre_body(xin_ref, pos_ref, wi_ref, bi_ref, wl_ref, bl_ref, ws_ref,
              bs_ref, wd_ref, bd_ref, ot_ref, oa_ref, *, m, d, pool,
              din_pad):
    x = xin_ref[...]
    if pool:
        x = jnp.max(x.reshape(m, _K, din_pad), axis=1)
        if din_pad > d:
            x = x[:, :d]
    h = jnp.maximum(
        jnp.dot(x, wi_ref[...], preferred_element_type=jnp.float32)
        + bi_ref[...], 0.0)
    v = jnp.dot(h, wl_ref[...], preferred_element_type=jnp.float32) + bl_ref[...]
    a_src = jnp.dot(h, ws_ref[...], preferred_element_type=jnp.float32) + bs_ref[...]
    a_dst = jnp.dot(h, wd_ref[...], preferred_element_type=jnp.float32) + bd_ref[...]
    dt = _pad_up(2 * d + 3)
    ot_ref[...] = jnp.concatenate(
        [v, a_src, pos_ref[...],
         jnp.zeros((m, dt - 2 * d - 3), jnp.float32)], axis=1)
    oa_ref[...] = a_dst


def _pre(p, xin, pos3, m, d, pool, din_pad):
    dt = _pad_up(2 * d + 3)
    r1 = lambda a: a.reshape(1, -1)
    return pl.pallas_call(
        functools.partial(_pre_body, m=m, d=d, pool=pool, din_pad=din_pad),
        out_shape=[jax.ShapeDtypeStruct((m, dt), jnp.float32),
                   jax.ShapeDtypeStruct((m, d), jnp.float32)],
    )(xin, pos3, p["lin_in"]["W"], r1(p["lin_in"]["b"]),
      p["lin"]["W"], r1(p["lin"]["b"]),
      p["lin_src"]["W"], r1(p["lin_src"]["b"]),
      p["lin_dst"]["W"], r1(p["lin_dst"]["b"]))


def _post_body(g_ref, ad_ref, pos_ref, w1p_ref, b1p_ref, w2p_ref, b2p_ref,
               w1a_ref, b1a_ref, w2a_ref, b2a_ref, wo_ref, bo_ref, o_ref, *,
               r, d, nb, final_mean):
    g = g_ref[...]
    vj = g[:, :d]
    aj = g[:, d:2 * d]
    pj = g[:, 2 * d:2 * d + 3]
    pd = (pos_ref[...][:, None, :] - pj.reshape(r, _K, 3)).reshape(r * _K, 3)
    t = jnp.maximum(
        jnp.dot(pd, w1p_ref[...], preferred_element_type=jnp.float32)
        + b1p_ref[...], 0.0)
    delta = jnp.maximum(
        jnp.dot(t, w2p_ref[...], preferred_element_type=jnp.float32)
        + b2p_ref[...], 0.0)
    d3 = delta.reshape(r, _K, d)
    al = (ad_ref[...][:, None, :] - aj.reshape(r, _K, d) + d3).reshape(r * _K, d)
    t2 = jnp.maximum(
        jnp.dot(al, w1a_ref[...], preferred_element_type=jnp.float32)
        + b1a_ref[...], 0.0)
    alpha = jnp.maximum(
        jnp.dot(t2, w2a_ref[...], preferred_element_type=jnp.float32)
        + b2a_ref[...], 0.0).reshape(r, _K, d)
    mx = jnp.max(alpha, axis=1, keepdims=True)
    e = jnp.exp(alpha - mx)
    w = e / jnp.sum(e, axis=1, keepdims=True)
    outv = jnp.sum(w * (vj.reshape(r, _K, d) + d3), axis=1)
    y = jnp.maximum(
        jnp.dot(outv, wo_ref[...], preferred_element_type=jnp.float32)
        + bo_ref[...], 0.0)
    if final_mean:
        o_ref[...] = jnp.mean(y.reshape(nb, r // nb, d), axis=1)
    else:
        o_ref[...] = y


def _post(p, g, a_dst, pos3, m, d, nb, final_mean):
    dt = _pad_up(2 * d + 3)
    r = min(512, m)
    ng = m // r
    if final_mean:
        r, ng = m, 1
        out_shape = jax.ShapeDtypeStruct((nb, d), jnp.float32)
        out_spec = pl.BlockSpec((nb, d), lambda i: (0, 0))
    else:
        out_shape = jax.ShapeDtypeStruct((m, d), jnp.float32)
        out_spec = pl.BlockSpec((r, d), lambda i: (i, 0))
    full = lambda s: pl.BlockSpec(s, lambda i: tuple(0 for _ in s))
    r1 = lambda a: a.reshape(1, -1)
    return pl.pallas_call(
        functools.partial(_post_body, r=r, d=d, nb=nb, final_mean=final_mean),
        grid=(ng,),
        in_specs=[pl.BlockSpec((r * _K, dt), lambda i: (i, 0)),
                  pl.BlockSpec((r, d), lambda i: (i, 0)),
                  pl.BlockSpec((r, 3), lambda i: (i, 0)),
                  full((3, 64)), full((1, 64)), full((64, d)), full((1, d)),
                  full((d, 64)), full((1, 64)), full((64, d)), full((1, d)),
                  full((d, d)), full((1, d))],
        out_specs=out_spec,
        out_shape=out_shape,
    )(g, a_dst, pos3,
      p["pos_nn"]["l1"]["W"], r1(p["pos_nn"]["l1"]["b"]),
      p["pos_nn"]["l2"]["W"], r1(p["pos_nn"]["l2"]["b"]),
      p["attn_nn"]["l1"]["W"], r1(p["attn_nn"]["l1"]["b"]),
      p["attn_nn"]["l2"]["W"], r1(p["attn_nn"]["l2"]["b"]),
      p["lin_out"]["W"], r1(p["lin_out"]["b"]))


def _tblock(p, xin, pos3, nbr, m, d, nb, pool, final_mean, din_pad=None):
    table, a_dst = _pre(p, xin, pos3, m, d, pool,
                        d if din_pad is None else din_pad)
    g = _sc_gather(table, nbr)
    return _post(p, g, a_dst, pos3, m, d, nb, final_mean)


def kernel(data, params):
    b, n, _ = data.shape
    pos3 = data.reshape(b * n, 3)
    p3 = (data[:, :, 0].reshape(b, 1, n), data[:, :, 1].reshape(b, 1, n),
          data[:, :, 2].reshape(b, 1, n))

    x = _bn_linear(pos3, params["mlp_input"])
    nbr = _knn(p3, p3, self_ex=True)
    x = _tblock(params["t_in"], x, pos3, nbr, b * n, _DIMS[0], b,
                pool=False, final_mean=False)

    for i in range(3):
        n_sub = n // 4
        d_out = _DIMS[i + 1]
        sx, sy, sz = _fps(*p3, n_sub)
        q3 = (sx.T.reshape(b, 1, n_sub), sy.T.reshape(b, 1, n_sub),
              sz.T.reshape(b, 1, n_sub))
        sub_pos3 = jnp.stack([sx.T, sy.T, sz.T], axis=-1).reshape(b * n_sub, 3)
        nbr_q = _knn(q3, p3, self_ex=False)
        d_pad = _pad_up(d_out)
        xm = _bn_linear(x, params["td"][i], pad_to=d_pad)
        gm = _sc_gather(xm, nbr_q)
        nbr_s = _knn(q3, q3, self_ex=True)
        x = _tblock(params["tb"][i], gm, sub_pos3, nbr_s, b * n_sub, d_out,
                    b, pool=True, final_mean=(i == 2), din_pad=d_pad)
        p3, n, pos3 = q3, n_sub, sub_pos3

    return x
